# Initial kernel scaffold; baseline (speedup 1.0000x reference)
import jax
import jax.numpy as jnp
from jax import lax
from jax.experimental import pallas as pl
from jax.experimental.pallas import tpu as pltpu
from jax.experimental.pallas import tpu_sc as plsc

B = 16384
D = 60
L = 20
NC = 2
NS = 16
NWKR = NC * NS
EPT = B // NWKR
G = 16
NCHUNK = EPT // G


def _rsqrt(a):
    i = plsc.bitcast(a, jnp.int32)
    i = 0x5F3759DF - lax.shift_right_arithmetic(i, 1)
    y = plsc.bitcast(i, jnp.float32)
    for _ in range(3):
        y = y * (1.5 - 0.5 * a * y * y)
    return y


def _sqrt(a):
    return a * _rsqrt(a)


def _body(head_hbm, rel_hbm, tail_hbm, hw_hbm, rw_hbm, tw_hbm,
          ent_hbm, rel_emb_hbm, word_hbm, proj_hbm, out_hbm,
          eh_idx, er_idx, et_idx, whw_idx, wrw_idx, wtw_idx,
          h_rows, t_rows, r_rows, p_rows, hw_rows, rw_rows, tw_rows,
          u_buf, pb_buf, scores_v, sem):
    wid = lax.axis_index("s") * NC + lax.axis_index("c")
    base = wid * EPT

    pltpu.sync_copy(head_hbm.at[pl.ds(base, EPT)], eh_idx)
    pltpu.sync_copy(rel_hbm.at[pl.ds(base, EPT)], er_idx)
    pltpu.sync_copy(tail_hbm.at[pl.ds(base, EPT)], et_idx)
    pltpu.sync_copy(hw_hbm.at[pl.ds(base * L, EPT * L)], whw_idx)
    pltpu.sync_copy(rw_hbm.at[pl.ds(base * L, EPT * L)], wrw_idx)
    pltpu.sync_copy(tw_hbm.at[pl.ds(base * L, EPT * L)], wtw_idx)

    iota = lax.iota(jnp.int32, 16)
    wrow0 = iota * L
    inv_l = jnp.float32(1.0 / L)
    zero16 = jnp.zeros((16,), jnp.float32)

    def chunk_body(c, _):
        eb = c * G
        descs = [
            pltpu.async_copy(ent_hbm.at[eh_idx.at[pl.ds(eb, G)]], h_rows, sem),
            pltpu.async_copy(ent_hbm.at[et_idx.at[pl.ds(eb, G)]], t_rows, sem),
            pltpu.async_copy(rel_emb_hbm.at[er_idx.at[pl.ds(eb, G)]], r_rows, sem),
            pltpu.async_copy(proj_hbm.at[er_idx.at[pl.ds(eb, G)]], p_rows, sem),
        ]
        for j in range(4):
            o = j * 80
            descs.append(pltpu.async_copy(
                word_hbm.at[whw_idx.at[pl.ds(eb * L + o, 80)]],
                hw_rows.at[pl.ds(o, 80)], sem))
            descs.append(pltpu.async_copy(
                word_hbm.at[wrw_idx.at[pl.ds(eb * L + o, 80)]],
                rw_rows.at[pl.ds(o, 80)], sem))
            descs.append(pltpu.async_copy(
                word_hbm.at[wtw_idx.at[pl.ds(eb * L + o, 80)]],
                tw_rows.at[pl.ds(o, 80)], sem))
        for dsc in descs:
            dsc.wait()

        def d_body(d, carry):
            pp, ph, pt = carry
            col = jnp.full((16,), 0, jnp.int32) + d
            p_d = plsc.load_gather(p_rows, [iota, col])
            h_d = plsc.load_gather(h_rows, [iota, col])
            t_d = plsc.load_gather(t_rows, [iota, col])
            r_d = plsc.load_gather(r_rows, [iota, col])
            hs0 = zero16; hs1 = zero16
            rs0 = zero16; rs1 = zero16
            ts0 = zero16; ts1 = zero16
            for l in range(0, L, 2):
                hs0 = hs0 + plsc.load_gather(hw_rows, [wrow0 + l, col])
                hs1 = hs1 + plsc.load_gather(hw_rows, [wrow0 + (l + 1), col])
                rs0 = rs0 + plsc.load_gather(rw_rows, [wrow0 + l, col])
                rs1 = rs1 + plsc.load_gather(rw_rows, [wrow0 + (l + 1), col])
                ts0 = ts0 + plsc.load_gather(tw_rows, [wrow0 + l, col])
                ts1 = ts1 + plsc.load_gather(tw_rows, [wrow0 + (l + 1), col])
            h_e = h_d + (hs0 + hs1) * inv_l
            r_e = r_d + (rs0 + rs1) * inv_l
            t_e = t_d + (ts0 + ts1) * inv_l
            u_d = h_e + r_e - t_e
            sidx = iota + d * 16
            plsc.store_scatter(u_buf, [sidx], u_d)
            plsc.store_scatter(pb_buf, [sidx], p_d)
            return (pp + p_d * p_d, ph + p_d * h_e, pt + p_d * t_e)

        pp, ph, pt = lax.fori_loop(0, D, d_body, (zero16, zero16, zero16))

        norm = _sqrt(pp)
        m = jnp.maximum(norm, jnp.float32(1e-12))
        cvec = (ph - pt) / (m * m)

        def d2_body(d, ss):
            sidx = iota + d * 16
            u = plsc.load_gather(u_buf, [sidx])
            p = plsc.load_gather(pb_buf, [sidx])
            v = u - cvec * p
            return ss + v * v

        ss = lax.fori_loop(0, D, d2_body, zero16)
        scores_v[...] = -_sqrt(ss)
        pltpu.sync_copy(scores_v, out_hbm.at[pl.ds(base + eb, G)])
        return 0

    lax.fori_loop(0, NCHUNK, chunk_body, 0)


@jax.jit
def _transh_sc(head, relation, tail, hw_flat, rw_flat, tw_flat,
               entity_embedding, relation_embedding, word_embedding,
               relation_projection):
    mesh = plsc.VectorSubcoreMesh(core_axis_name="c", subcore_axis_name="s")
    f = pl.kernel(
        _body,
        out_type=jax.ShapeDtypeStruct((B,), jnp.float32),
        mesh=mesh,
        scratch_types=[
            pltpu.VMEM((EPT,), jnp.int32),
            pltpu.VMEM((EPT,), jnp.int32),
            pltpu.VMEM((EPT,), jnp.int32),
            pltpu.VMEM((EPT * L,), jnp.int32),
            pltpu.VMEM((EPT * L,), jnp.int32),
            pltpu.VMEM((EPT * L,), jnp.int32),
            pltpu.VMEM((G, D), jnp.float32),
            pltpu.VMEM((G, D), jnp.float32),
            pltpu.VMEM((G, D), jnp.float32),
            pltpu.VMEM((G, D), jnp.float32),
            pltpu.VMEM((G * L, D), jnp.float32),
            pltpu.VMEM((G * L, D), jnp.float32),
            pltpu.VMEM((G * L, D), jnp.float32),
            pltpu.VMEM((G * D,), jnp.float32),
            pltpu.VMEM((G * D,), jnp.float32),
            pltpu.VMEM((G,), jnp.float32),
            pltpu.SemaphoreType.DMA,
        ],
    )
    return f(head, relation, tail, hw_flat, rw_flat, tw_flat,
             entity_embedding, relation_embedding, word_embedding,
             relation_projection)


def kernel(head, relation, tail, head_w, rel_w, tail_w,
           entity_embedding, relation_embedding, word_embedding,
           relation_projection):
    i32 = jnp.int32
    return _transh_sc(
        head.astype(i32), relation.astype(i32), tail.astype(i32),
        head_w.astype(i32).reshape(-1), rel_w.astype(i32).reshape(-1),
        tail_w.astype(i32).reshape(-1),
        entity_embedding, relation_embedding, word_embedding,
        relation_projection)


# speedup vs baseline: 1.9182x; 1.9182x over previous
import jax
import jax.numpy as jnp
from jax import lax
from jax.experimental import pallas as pl
from jax.experimental.pallas import tpu as pltpu
from jax.experimental.pallas import tpu_sc as plsc

B = 16384
D = 60
DP = 64
L = 20
NC = 2
NS = 16
NWKR = NC * NS
EPT = B // NWKR
G = 16
NCHUNK = EPT // G


def _rsqrt(a):
    i = plsc.bitcast(a, jnp.int32)
    i = 0x5F3759DF - lax.shift_right_arithmetic(i, 1)
    y = plsc.bitcast(i, jnp.float32)
    for _ in range(3):
        y = y * (1.5 - 0.5 * a * y * y)
    return y


def _sqrt(a):
    return a * _rsqrt(a)


def _body(head_hbm, rel_hbm, tail_hbm, hw_hbm, rw_hbm, tw_hbm,
          ent_hbm, rel_emb_hbm, word_hbm, proj_hbm, out_hbm,
          eh_idx, er_idx, et_idx, whw_idx, wrw_idx, wtw_idx,
          h_rows, t_rows, r_rows, p_rows, hw_rows, rw_rows, tw_rows,
          u_buf, pb_buf, scores_v, sem):
    wid = lax.axis_index("s") * NC + lax.axis_index("c")
    base = wid * EPT

    pltpu.sync_copy(head_hbm.at[pl.ds(base, EPT)], eh_idx)
    pltpu.sync_copy(rel_hbm.at[pl.ds(base, EPT)], er_idx)
    pltpu.sync_copy(tail_hbm.at[pl.ds(base, EPT)], et_idx)
    pltpu.sync_copy(hw_hbm.at[pl.ds(base * L, EPT * L)], whw_idx)
    pltpu.sync_copy(rw_hbm.at[pl.ds(base * L, EPT * L)], wrw_idx)
    pltpu.sync_copy(tw_hbm.at[pl.ds(base * L, EPT * L)], wtw_idx)

    iota = lax.iota(jnp.int32, 16)
    wrow0 = iota * L
    inv_l = jnp.float32(1.0 / L)
    zero16 = jnp.zeros((16,), jnp.float32)

    def chunk_body(c, _):
        eb = c * G
        descs = []
        e_h = eh_idx[pl.ds(eb, G)]
        e_t = et_idx[pl.ds(eb, G)]
        for k in range(G):
            descs.append(pltpu.async_copy(
                ent_hbm.at[pl.ds(e_h[k], 1)], h_rows.at[pl.ds(k, 1)], sem))
            descs.append(pltpu.async_copy(
                ent_hbm.at[pl.ds(e_t[k], 1)], t_rows.at[pl.ds(k, 1)], sem))
        descs.append(pltpu.async_copy(
            rel_emb_hbm.at[er_idx.at[pl.ds(eb, G)]], r_rows, sem))
        descs.append(pltpu.async_copy(
            proj_hbm.at[er_idx.at[pl.ds(eb, G)]], p_rows, sem))
        for j in range(4):
            o = j * 80
            descs.append(pltpu.async_copy(
                word_hbm.at[whw_idx.at[pl.ds(eb * L + o, 80)]],
                hw_rows.at[pl.ds(o, 80)], sem))
            descs.append(pltpu.async_copy(
                word_hbm.at[wrw_idx.at[pl.ds(eb * L + o, 80)]],
                rw_rows.at[pl.ds(o, 80)], sem))
            descs.append(pltpu.async_copy(
                word_hbm.at[wtw_idx.at[pl.ds(eb * L + o, 80)]],
                tw_rows.at[pl.ds(o, 80)], sem))
        for dsc in descs:
            dsc.wait()

        def d_body(d, carry):
            pp, ph, pt = carry
            col = jnp.full((16,), 0, jnp.int32) + d
            p_d = plsc.load_gather(p_rows, [iota, col])
            h_d = plsc.load_gather(h_rows, [iota, col])
            t_d = plsc.load_gather(t_rows, [iota, col])
            r_d = plsc.load_gather(r_rows, [iota, col])
            hs0 = zero16; hs1 = zero16
            rs0 = zero16; rs1 = zero16
            ts0 = zero16; ts1 = zero16
            for l in range(0, L, 2):
                hs0 = hs0 + plsc.load_gather(hw_rows, [wrow0 + l, col])
                hs1 = hs1 + plsc.load_gather(hw_rows, [wrow0 + (l + 1), col])
                rs0 = rs0 + plsc.load_gather(rw_rows, [wrow0 + l, col])
                rs1 = rs1 + plsc.load_gather(rw_rows, [wrow0 + (l + 1), col])
                ts0 = ts0 + plsc.load_gather(tw_rows, [wrow0 + l, col])
                ts1 = ts1 + plsc.load_gather(tw_rows, [wrow0 + (l + 1), col])
            h_e = h_d + (hs0 + hs1) * inv_l
            r_e = r_d + (rs0 + rs1) * inv_l
            t_e = t_d + (ts0 + ts1) * inv_l
            u_d = h_e + r_e - t_e
            sidx = iota + d * 16
            plsc.store_scatter(u_buf, [sidx], u_d)
            plsc.store_scatter(pb_buf, [sidx], p_d)
            return (pp + p_d * p_d, ph + p_d * h_e, pt + p_d * t_e)

        pp, ph, pt = lax.fori_loop(0, D, d_body, (zero16, zero16, zero16))

        norm = _sqrt(pp)
        m = jnp.maximum(norm, jnp.float32(1e-12))
        cvec = (ph - pt) / (m * m)

        def d2_body(d, ss):
            sidx = iota + d * 16
            u = plsc.load_gather(u_buf, [sidx])
            p = plsc.load_gather(pb_buf, [sidx])
            v = u - cvec * p
            return ss + v * v

        ss = lax.fori_loop(0, D, d2_body, zero16)
        scores_v[...] = -_sqrt(ss)
        pltpu.sync_copy(scores_v, out_hbm.at[pl.ds(base + eb, G)])
        return 0

    lax.fori_loop(0, NCHUNK, chunk_body, 0)


SCRATCH = [
    pltpu.VMEM((EPT,), jnp.int32),
    pltpu.VMEM((EPT,), jnp.int32),
    pltpu.VMEM((EPT,), jnp.int32),
    pltpu.VMEM((EPT * L,), jnp.int32),
    pltpu.VMEM((EPT * L,), jnp.int32),
    pltpu.VMEM((EPT * L,), jnp.int32),
    pltpu.VMEM((G, D), jnp.float32),
    pltpu.VMEM((G, D), jnp.float32),
    pltpu.VMEM((G, DP), jnp.float32),
    pltpu.VMEM((G, DP), jnp.float32),
    pltpu.VMEM((G * L, DP), jnp.float32),
    pltpu.VMEM((G * L, DP), jnp.float32),
    pltpu.VMEM((G * L, DP), jnp.float32),
    pltpu.VMEM((G * D,), jnp.float32),
    pltpu.VMEM((G * D,), jnp.float32),
    pltpu.VMEM((G,), jnp.float32),
    pltpu.SemaphoreType.DMA,
]


@jax.jit
def _transh_sc(head, relation, tail, hw_flat, rw_flat, tw_flat,
               entity_embedding, rel_emb_p, word_p, proj_p):
    mesh = plsc.VectorSubcoreMesh(core_axis_name="c", subcore_axis_name="s")
    f = pl.kernel(
        _body,
        out_type=jax.ShapeDtypeStruct((B,), jnp.float32),
        mesh=mesh,
        compiler_params=pltpu.CompilerParams(
            needs_layout_passes=False, use_tc_tiling_on_sc=False),
        scratch_types=SCRATCH,
    )
    return f(head, relation, tail, hw_flat, rw_flat, tw_flat,
             entity_embedding, rel_emb_p, word_p, proj_p)


def kernel(head, relation, tail, head_w, rel_w, tail_w,
           entity_embedding, relation_embedding, word_embedding,
           relation_projection):
    i32 = jnp.int32
    pad = ((0, 0), (0, DP - D))
    return _transh_sc(
        head.astype(i32), relation.astype(i32), tail.astype(i32),
        head_w.astype(i32).reshape(-1), rel_w.astype(i32).reshape(-1),
        tail_w.astype(i32).reshape(-1),
        entity_embedding,
        jnp.pad(relation_embedding, pad),
        jnp.pad(word_embedding, pad),
        jnp.pad(relation_projection, pad))


# speedup vs baseline: 3.5006x; 1.8250x over previous
import jax
import jax.numpy as jnp
from jax import lax
from jax.experimental import pallas as pl
from jax.experimental.pallas import tpu as pltpu
from jax.experimental.pallas import tpu_sc as plsc

B = 16384
D = 60
DP = 64
NWIN = DP // 16
L = 20
NC = 2
NS = 16
NWKR = NC * NS
EPT = B // NWKR
G = 16
NCHUNK = EPT // G


def _rsqrt(a):
    i = plsc.bitcast(a, jnp.int32)
    i = 0x5F3759DF - lax.shift_right_arithmetic(i, 1)
    y = plsc.bitcast(i, jnp.float32)
    for _ in range(3):
        y = y * (1.5 - 0.5 * a * y * y)
    return y


def _sqrt(a):
    return a * _rsqrt(a)


def _body(head_hbm, rel_hbm, tail_hbm, hw_hbm, rw_hbm, tw_hbm,
          ent_hbm, rel_emb_hbm, word_hbm, proj_hbm, out_hbm,
          eh_idx, er_idx, et_idx, whw_idx, wrw_idx, wtw_idx,
          h_rows, t_rows, r_rows, p_rows, hw_rows, rw_rows, tw_rows,
          scores_v, sem):
    wid = lax.axis_index("s") * NC + lax.axis_index("c")
    base = wid * EPT

    pltpu.sync_copy(head_hbm.at[pl.ds(base, EPT)], eh_idx)
    pltpu.sync_copy(rel_hbm.at[pl.ds(base, EPT)], er_idx)
    pltpu.sync_copy(tail_hbm.at[pl.ds(base, EPT)], et_idx)
    pltpu.sync_copy(hw_hbm.at[pl.ds(base * L, EPT * L)], whw_idx)
    pltpu.sync_copy(rw_hbm.at[pl.ds(base * L, EPT * L)], wrw_idx)
    pltpu.sync_copy(tw_hbm.at[pl.ds(base * L, EPT * L)], wtw_idx)

    iota = lax.iota(jnp.int32, 16)
    inv_l = jnp.float32(1.0 / L)
    zero16 = jnp.zeros((16,), jnp.float32)
    iota_c12 = jnp.minimum(iota, 11) + 48
    m12 = jnp.where(iota < 12, jnp.float32(1.0), jnp.float32(0.0))

    def chunk_body(c, _):
        eb = c * G
        descs = []
        e_h = eh_idx[pl.ds(eb, G)]
        e_t = et_idx[pl.ds(eb, G)]
        for k in range(G):
            descs.append(pltpu.async_copy(
                ent_hbm.at[pl.ds(e_h[k], 1)], h_rows.at[pl.ds(k, 1)], sem))
            descs.append(pltpu.async_copy(
                ent_hbm.at[pl.ds(e_t[k], 1)], t_rows.at[pl.ds(k, 1)], sem))
        descs.append(pltpu.async_copy(
            rel_emb_hbm.at[er_idx.at[pl.ds(eb, G)]], r_rows, sem))
        descs.append(pltpu.async_copy(
            proj_hbm.at[er_idx.at[pl.ds(eb, G)]], p_rows, sem))
        for j in range(4):
            o = j * 80
            descs.append(pltpu.async_copy(
                word_hbm.at[whw_idx.at[pl.ds(eb * L + o, 80)]],
                hw_rows.at[pl.ds(o, 80)], sem))
            descs.append(pltpu.async_copy(
                word_hbm.at[wrw_idx.at[pl.ds(eb * L + o, 80)]],
                rw_rows.at[pl.ds(o, 80)], sem))
            descs.append(pltpu.async_copy(
                word_hbm.at[wtw_idx.at[pl.ds(eb * L + o, 80)]],
                tw_rows.at[pl.ds(o, 80)], sem))
        for dsc in descs:
            dsc.wait()

        def elem_body(i, ss_acc):
            rb = i * L
            i_s = jnp.full((16,), 0, jnp.int32) + i
            hv = [h_rows[i, pl.ds(w * 16, 16)] for w in range(NWIN - 1)]
            hv.append(plsc.load_gather(h_rows, [i_s, iota_c12]) * m12)
            tv = [t_rows[i, pl.ds(w * 16, 16)] for w in range(NWIN - 1)]
            tv.append(plsc.load_gather(t_rows, [i_s, iota_c12]) * m12)
            rv = [r_rows[i, pl.ds(w * 16, 16)] for w in range(NWIN)]
            pv = [p_rows[i, pl.ds(w * 16, 16)] for w in range(NWIN)]
            hs = [zero16] * NWIN
            rs = [zero16] * NWIN
            ts = [zero16] * NWIN
            for l in range(L):
                r = rb + l
                for w in range(NWIN):
                    o = w * 16
                    hs[w] = hs[w] + hw_rows[r, pl.ds(o, 16)]
                    rs[w] = rs[w] + rw_rows[r, pl.ds(o, 16)]
                    ts[w] = ts[w] + tw_rows[r, pl.ds(o, 16)]
            he = [hv[w] + hs[w] * inv_l for w in range(NWIN)]
            re = [rv[w] + rs[w] * inv_l for w in range(NWIN)]
            te = [tv[w] + ts[w] * inv_l for w in range(NWIN)]
            ppv = pv[0] * pv[0]
            phv = pv[0] * he[0]
            ptv = pv[0] * te[0]
            for w in range(1, NWIN):
                ppv = ppv + pv[w] * pv[w]
                phv = phv + pv[w] * he[w]
                ptv = ptv + pv[w] * te[w]
            pp = jnp.sum(ppv)
            ph = jnp.sum(phv)
            pt = jnp.sum(ptv)
            pp_s = jnp.zeros((16,), jnp.float32) + pp
            m = jnp.maximum(_sqrt(pp_s), jnp.float32(1e-12))
            cv = (jnp.zeros((16,), jnp.float32) + (ph - pt)) / (m * m)
            ssv = zero16
            for w in range(NWIN):
                v = he[w] + re[w] - te[w] - cv * pv[w]
                ssv = ssv + v * v
            ss = jnp.sum(ssv)
            return jnp.where(iota == i, jnp.zeros((16,), jnp.float32) + ss,
                             ss_acc)

        ss_acc = lax.fori_loop(0, G, elem_body, zero16)
        scores_v[...] = -_sqrt(ss_acc)
        pltpu.sync_copy(scores_v, out_hbm.at[pl.ds(base + eb, G)])
        return 0

    lax.fori_loop(0, NCHUNK, chunk_body, 0)


SCRATCH = [
    pltpu.VMEM((EPT,), jnp.int32),
    pltpu.VMEM((EPT,), jnp.int32),
    pltpu.VMEM((EPT,), jnp.int32),
    pltpu.VMEM((EPT * L,), jnp.int32),
    pltpu.VMEM((EPT * L,), jnp.int32),
    pltpu.VMEM((EPT * L,), jnp.int32),
    pltpu.VMEM((G, D), jnp.float32),
    pltpu.VMEM((G, D), jnp.float32),
    pltpu.VMEM((G, DP), jnp.float32),
    pltpu.VMEM((G, DP), jnp.float32),
    pltpu.VMEM((G * L, DP), jnp.float32),
    pltpu.VMEM((G * L, DP), jnp.float32),
    pltpu.VMEM((G * L, DP), jnp.float32),
    pltpu.VMEM((G,), jnp.float32),
    pltpu.SemaphoreType.DMA,
]


@jax.jit
def _transh_sc(head, relation, tail, hw_flat, rw_flat, tw_flat,
               entity_embedding, rel_emb_p, word_p, proj_p):
    mesh = plsc.VectorSubcoreMesh(core_axis_name="c", subcore_axis_name="s")
    f = pl.kernel(
        _body,
        out_type=jax.ShapeDtypeStruct((B,), jnp.float32),
        mesh=mesh,
        compiler_params=pltpu.CompilerParams(
            needs_layout_passes=False, use_tc_tiling_on_sc=False),
        scratch_types=SCRATCH,
    )
    return f(head, relation, tail, hw_flat, rw_flat, tw_flat,
             entity_embedding, rel_emb_p, word_p, proj_p)


def kernel(head, relation, tail, head_w, rel_w, tail_w,
           entity_embedding, relation_embedding, word_embedding,
           relation_projection):
    i32 = jnp.int32
    pad = ((0, 0), (0, DP - D))
    return _transh_sc(
        head.astype(i32), relation.astype(i32), tail.astype(i32),
        head_w.astype(i32).reshape(-1), rel_w.astype(i32).reshape(-1),
        tail_w.astype(i32).reshape(-1),
        entity_embedding,
        jnp.pad(relation_embedding, pad),
        jnp.pad(word_embedding, pad),
        jnp.pad(relation_projection, pad))
